# Initial kernel scaffold; baseline (speedup 1.0000x reference)
import jax
import jax.numpy as jnp
from jax.experimental import pallas as pl


def kernel(g, t):
    raise NotImplementedError("write your pallas kernel here")


# speedup vs baseline: 1.7516x; 1.7516x over previous
import jax
import jax.numpy as jnp
import numpy as np
from jax.experimental import pallas as pl
from jax.experimental.pallas import tpu as pltpu

_PI = float(np.arccos(0.0) * 2.0)
_D225 = _PI / 8
_D675 = 3 * _PI / 8
_D1125 = 5 * _PI / 8
_D1575 = 7 * _PI / 8
_D180 = _PI


def _nms_body(g_ref, t_ref, o_ref):
    g = g_ref[0]
    t = jnp.abs(t_ref[0])

    left = jnp.concatenate([g[:, :1], g[:, :-1]], axis=1)
    right = jnp.concatenate([g[:, 1:], g[:, -1:]], axis=1)

    def up(x):
        return jnp.concatenate([x[:1], x[:-1]], axis=0)

    def down(x):
        return jnp.concatenate([x[1:], x[-1:]], axis=0)

    top = up(g)
    bottom = down(g)
    top_left = up(left)
    top_right = up(right)
    bottom_left = down(left)
    bottom_right = down(right)

    m1 = t < _D225
    m2 = t < _D675
    m3 = t < _D1125
    m4 = t < _D1575

    cond_h = (g >= left) & (g >= right)
    cond_d45 = (g >= top_right) & (g >= bottom_left)
    cond_v = (g >= top) & (g >= bottom)
    cond_d135 = (g >= top_left) & (g >= bottom_right)

    h_sel = m1 | (~m4 & (t <= _D180))
    d45_sel = ~m1 & m2
    v_sel = ~m2 & m3
    d135_sel = ~m3 & m4

    keep = ((h_sel & cond_h) | (d45_sel & cond_d45)
            | (v_sel & cond_v) | (d135_sel & cond_d135))
    o_ref[0] = jnp.where(keep, g, jnp.zeros_like(g))


def kernel(g, t):
    B = g.shape[0]
    H, W = g.shape[2], g.shape[3]
    g3 = g.reshape(B, H, W)
    t3 = t.reshape(B, H, W)
    out = pl.pallas_call(
        _nms_body,
        grid=(B,),
        in_specs=[
            pl.BlockSpec((1, H, W), lambda i: (i, 0, 0)),
            pl.BlockSpec((1, H, W), lambda i: (i, 0, 0)),
        ],
        out_specs=pl.BlockSpec((1, H, W), lambda i: (i, 0, 0)),
        out_shape=jax.ShapeDtypeStruct((B, H, W), jnp.float32),
    )(g3, t3)
    return out.reshape(B, 1, H, W)


# speedup vs baseline: 3.3476x; 1.9111x over previous
import jax
import jax.numpy as jnp
import numpy as np
from jax.experimental import pallas as pl
from jax.experimental.pallas import tpu as pltpu

_PI = float(np.arccos(0.0) * 2.0)
_D225 = _PI / 8
_D675 = 3 * _PI / 8
_D1125 = 5 * _PI / 8
_D1575 = 7 * _PI / 8
_D180 = _PI


def _nms_body(g_ref, t_ref, o_ref):
    g = g_ref[0]
    t = jnp.abs(t_ref[0])

    left = jnp.concatenate([g[:, :1], g[:, :-1]], axis=1)
    right = jnp.concatenate([g[:, 1:], g[:, -1:]], axis=1)

    def up(x):
        return jnp.concatenate([x[:1], x[:-1]], axis=0)

    def down(x):
        return jnp.concatenate([x[1:], x[-1:]], axis=0)

    top = up(g)
    bottom = down(g)
    top_left = up(left)
    top_right = up(right)
    bottom_left = down(left)
    bottom_right = down(right)

    pair_h = jnp.maximum(left, right)
    pair_d45 = jnp.maximum(top_right, bottom_left)
    pair_v = jnp.maximum(top, bottom)
    pair_d135 = jnp.maximum(top_left, bottom_right)

    hm = (t < _D225) | (t >= _D1575)
    nmax = jnp.where(
        hm, pair_h,
        jnp.where(t < _D675, pair_d45,
                  jnp.where(t < _D1125, pair_v, pair_d135)))

    keep = (g >= nmax) & (t <= _D180)
    o_ref[0] = jnp.where(keep, g, jnp.zeros_like(g))


def kernel(g, t):
    B = g.shape[0]
    H, W = g.shape[2], g.shape[3]
    g3 = g.reshape(B, H, W)
    t3 = t.reshape(B, H, W)
    out = pl.pallas_call(
        _nms_body,
        grid=(B,),
        in_specs=[
            pl.BlockSpec((1, H, W), lambda i: (i, 0, 0)),
            pl.BlockSpec((1, H, W), lambda i: (i, 0, 0)),
        ],
        out_specs=pl.BlockSpec((1, H, W), lambda i: (i, 0, 0)),
        out_shape=jax.ShapeDtypeStruct((B, H, W), jnp.float32),
    )(g3, t3)
    return out.reshape(B, 1, H, W)
